# baseline (speedup 1.0000x reference)
import functools

import jax
import jax.numpy as jnp
from jax import lax
from jax.experimental import pallas as pl
from jax.experimental.pallas import tpu as pltpu
from jax.experimental.pallas import tpu_sc as plsc

DIM = 64
IDX_CHUNK = 128


def _proj_kernel(prompt_ref, wp_ref, bp_ref, wc_ref, out_ref):
    pe = (
        jnp.dot(prompt_ref[...], wp_ref[...], preferred_element_type=jnp.float32)
        + bp_ref[...]
    )
    nsq = jnp.sum(pe * pe)
    denom = jnp.maximum(jnp.sqrt(nsq), 1e-12)
    c = jnp.sum(pe * wc_ref[...]) / denom
    out_ref[...] = jnp.broadcast_to(c, (1, 16))


def _fast_rsqrt(x):
    i = plsc.bitcast(x, jnp.int32)
    i = 0x5F3759DF - lax.shift_right_logical(i, 1)
    y = plsc.bitcast(i, jnp.float32)
    for _ in range(3):
        y = y * (1.5 - 0.5 * x * y * y)
    return y


def _make_sc_kernel(batch, num_workers, bpw):
    n_chunks = bpw // IDX_CHUNK
    n_groups = bpw // 16
    mesh = plsc.VectorSubcoreMesh(core_axis_name="c", subcore_axis_name="s")

    @functools.partial(
        pl.kernel,
        mesh=mesh,
        out_type=jax.ShapeDtypeStruct((batch,), jnp.float32),
        compiler_params=pltpu.CompilerParams(
            needs_layout_passes=False, use_tc_tiling_on_sc=False
        ),
        scratch_types=[
            [pltpu.VMEM((IDX_CHUNK,), jnp.int32) for _ in range(n_chunks)],
            pltpu.VMEM((bpw, DIM), jnp.float32),
            pltpu.VMEM((DIM,), jnp.float32),
            pltpu.VMEM((16,), jnp.float32),
            pltpu.VMEM((bpw,), jnp.float32),
            pltpu.SemaphoreType.DMA,
        ],
    )
    def sc_kernel(ids_hbm, p_hbm, wc_hbm, c_hbm, out_hbm,
                  idx_vs, rows_v, wc_v, c_v, out_v, sem):
        wid = lax.axis_index("s") * 2 + lax.axis_index("c")
        base = wid * bpw
        pltpu.sync_copy(wc_hbm, wc_v)
        pltpu.sync_copy(c_hbm, c_v)
        for k in range(n_chunks):
            pltpu.sync_copy(
                ids_hbm.at[pl.ds(base + k * IDX_CHUNK, IDX_CHUNK)], idx_vs[k]
            )
        copies = []
        for k in range(n_chunks):
            copies.append(
                pltpu.async_copy(
                    p_hbm.at[idx_vs[k]],
                    rows_v.at[pl.ds(k * IDX_CHUNK, IDX_CHUNK)],
                    sem,
                )
            )
        for cp in copies:
            cp.wait()

        c_vec = c_v[...]
        lane = lax.iota(jnp.int32, 16)
        wc_chunks = [wc_v[pl.ds(jj * 16, 16)] for jj in range(DIM // 16)]

        def body(g, carry):
            ridx = lane + g * 16
            accd = jnp.zeros((16,), jnp.float32)
            accn = jnp.zeros((16,), jnp.float32)
            for j in range(DIM):
                cidx = jnp.full((16,), j, dtype=jnp.int32)
                col = plsc.load_gather(rows_v, [ridx, cidx])
                accd = accd + col * wc_chunks[j // 16][j % 16]
                accn = accn + col * col
            out_v[pl.ds(g * 16, 16)] = accd * _fast_rsqrt(accn) - c_vec
            return carry

        lax.fori_loop(0, n_groups, body, 0)
        pltpu.sync_copy(out_v, out_hbm.at[pl.ds(base, bpw)])

    return sc_kernel


def kernel(model_ids, prompt_embed, P, Wp, bp, Wc):
    batch = model_ids.shape[0]
    info = plsc.get_sparse_core_info()
    num_workers = info.num_cores * info.num_subcores
    bpw = batch // num_workers

    c_out = pl.pallas_call(
        _proj_kernel,
        out_shape=jax.ShapeDtypeStruct((1, 16), jnp.float32),
    )(prompt_embed, Wp, bp.reshape(1, DIM), Wc.reshape(1, DIM))
    c16 = c_out.reshape(16)

    ids = model_ids.astype(jnp.int32)
    wc_flat = Wc.reshape(DIM)
    out = _make_sc_kernel(batch, num_workers, bpw)(ids, P, wc_flat, c16)
    return out


# speedup vs baseline: 2.2660x; 2.2660x over previous
import functools

import jax
import jax.numpy as jnp
from jax import lax
from jax.experimental import pallas as pl
from jax.experimental.pallas import tpu as pltpu
from jax.experimental.pallas import tpu_sc as plsc

DIM = 64
TILE_ROWS = 8
CHUNK = 64


def _proj_kernel(prompt_ref, wp_ref, bp_ref, wc_ref, out_ref):
    pe = (
        jnp.dot(prompt_ref[...], wp_ref[...], preferred_element_type=jnp.float32)
        + bp_ref[...]
    )
    nsq = jnp.sum(pe * pe)
    denom = jnp.maximum(jnp.sqrt(nsq), 1e-12)
    c = jnp.sum(pe * wc_ref[...]) / denom
    out_ref[...] = jnp.broadcast_to(c, (1, 16))


def _fast_rsqrt(x):
    i = plsc.bitcast(x, jnp.int32)
    i = 0x5F3759DF - lax.shift_right_logical(i, 1)
    y = plsc.bitcast(i, jnp.float32)
    for _ in range(3):
        y = y * (1.5 - 0.5 * x * y * y)
    return y


def _make_sc_kernel(batch, num_workers, bpw):
    n_chunks = bpw // CHUNK
    groups_per_chunk = CHUNK // 16
    mesh = plsc.VectorSubcoreMesh(core_axis_name="c", subcore_axis_name="s")

    @functools.partial(
        pl.kernel,
        mesh=mesh,
        out_type=jax.ShapeDtypeStruct((batch,), jnp.float32),
        compiler_params=pltpu.CompilerParams(
            needs_layout_passes=False, use_tc_tiling_on_sc=True
        ),
        scratch_types=[
            pltpu.VMEM((bpw,), jnp.int32),
            pltpu.VMEM((CHUNK, TILE_ROWS, DIM), jnp.float32),
            pltpu.VMEM((DIM,), jnp.float32),
            pltpu.VMEM((16,), jnp.float32),
            pltpu.VMEM((bpw,), jnp.float32),
            pltpu.SemaphoreType.DMA,
        ],
    )
    def sc_kernel(ids_hbm, p_hbm, wc_hbm, c_hbm, out_hbm,
                  ids_v, tiles_v, wc_v, c_v, out_v, sem):
        wid = lax.axis_index("s") * 2 + lax.axis_index("c")
        base = wid * bpw
        pltpu.sync_copy(wc_hbm, wc_v)
        pltpu.sync_copy(c_hbm, c_v)
        pltpu.sync_copy(ids_hbm.at[pl.ds(base, bpw)], ids_v)

        c_vec = c_v[...]
        lane = lax.iota(jnp.int32, 16)
        wc_chunks = [wc_v[pl.ds(jj * 16, 16)] for jj in range(DIM // 16)]

        def chunk_body(k, carry):
            copies = []
            for g4 in range(groups_per_chunk):
                tv = lax.shift_right_logical(
                    ids_v[pl.ds(k * CHUNK + g4 * 16, 16)], 3
                )
                for j in range(16):
                    copies.append(
                        pltpu.async_copy(
                            p_hbm.at[pl.ds(tv[j], 1)],
                            tiles_v.at[pl.ds(g4 * 16 + j, 1)],
                            sem,
                        )
                    )
            for cp in copies:
                cp.wait()

            def grp_body(g, carry2):
                pos = k * CHUNK + g * 16
                idv = ids_v[pl.ds(pos, 16)]
                kidx = lane + g * 16
                ridx = lax.bitwise_and(idv, 7)
                accd = jnp.zeros((16,), jnp.float32)
                accn = jnp.zeros((16,), jnp.float32)
                for j in range(DIM):
                    cidx = jnp.full((16,), j, dtype=jnp.int32)
                    col = plsc.load_gather(tiles_v, [kidx, ridx, cidx])
                    accd = accd + col * wc_chunks[j // 16][j % 16]
                    accn = accn + col * col
                out_v[pl.ds(pos, 16)] = accd * _fast_rsqrt(accn) - c_vec
                return carry2

            lax.fori_loop(0, groups_per_chunk, grp_body, 0)
            return carry

        lax.fori_loop(0, n_chunks, chunk_body, 0)
        pltpu.sync_copy(out_v, out_hbm.at[pl.ds(base, bpw)])

    return sc_kernel


def kernel(model_ids, prompt_embed, P, Wp, bp, Wc):
    batch = model_ids.shape[0]
    num_models = P.shape[0]
    info = plsc.get_sparse_core_info()
    num_workers = info.num_cores * info.num_subcores
    bpw = batch // num_workers

    c_out = pl.pallas_call(
        _proj_kernel,
        out_shape=jax.ShapeDtypeStruct((1, 16), jnp.float32),
    )(prompt_embed, Wp, bp.reshape(1, DIM), Wc.reshape(1, DIM))
    c16 = c_out.reshape(16)

    ids = model_ids.astype(jnp.int32)
    p3 = P.reshape(num_models // TILE_ROWS, TILE_ROWS, DIM)
    wc_flat = Wc.reshape(DIM)
    out = _make_sc_kernel(batch, num_workers, bpw)(ids, p3, wc_flat, c16)
    return out


# speedup vs baseline: 4.9137x; 2.1685x over previous
import functools

import jax
import jax.numpy as jnp
from jax import lax
from jax.experimental import pallas as pl
from jax.experimental.pallas import tpu as pltpu
from jax.experimental.pallas import tpu_sc as plsc

DIM = 64
COL_BLOCK = 32768
IDX_CHUNK = 128


def _proj_kernel(prompt_ref, wp_ref, bp_ref, wc_ref, out_ref):
    pe = (
        jnp.dot(prompt_ref[...], wp_ref[...], preferred_element_type=jnp.float32)
        + bp_ref[...]
    )
    nsq = jnp.sum(pe * pe)
    denom = jnp.maximum(jnp.sqrt(nsq), 1e-12)
    c = jnp.sum(pe * wc_ref[...]) / denom
    out_ref[...] = jnp.broadcast_to(c, (1, 16))


def _reduce_kernel(pt_ref, wc_ref, c_ref, out_ref):
    x = pt_ref[...]
    wc = wc_ref[...]
    dot = jnp.sum(x * wc, axis=0)
    nsq = jnp.sum(x * x, axis=0)
    denom = jnp.maximum(jnp.sqrt(nsq), 1e-12)
    out_ref[...] = dot / denom - c_ref[0, 0]


def _make_sc_kernel(batch, num_workers, bpw):
    n_chunks = bpw // IDX_CHUNK
    mesh = plsc.VectorSubcoreMesh(core_axis_name="c", subcore_axis_name="s")

    @functools.partial(
        pl.kernel,
        mesh=mesh,
        out_type=jax.ShapeDtypeStruct((batch,), jnp.float32),
        compiler_params=pltpu.CompilerParams(
            needs_layout_passes=False, use_tc_tiling_on_sc=False
        ),
        scratch_types=[
            [pltpu.VMEM((IDX_CHUNK,), jnp.int32) for _ in range(n_chunks)],
            pltpu.VMEM((bpw,), jnp.float32),
            pltpu.SemaphoreType.DMA,
        ],
    )
    def sc_kernel(ids_hbm, r_hbm, out_hbm, idx_vs, out_v, sem):
        wid = lax.axis_index("s") * 2 + lax.axis_index("c")
        base = wid * bpw
        for k in range(n_chunks):
            pltpu.sync_copy(
                ids_hbm.at[pl.ds(base + k * IDX_CHUNK, IDX_CHUNK)], idx_vs[k]
            )
        copies = []
        for k in range(n_chunks):
            copies.append(
                pltpu.async_copy(
                    r_hbm.at[idx_vs[k]],
                    out_v.at[pl.ds(k * IDX_CHUNK, IDX_CHUNK)],
                    sem,
                )
            )
        for cp in copies:
            cp.wait()
        pltpu.sync_copy(out_v, out_hbm.at[pl.ds(base, bpw)])

    return sc_kernel


def kernel(model_ids, prompt_embed, P, Wp, bp, Wc):
    batch = model_ids.shape[0]
    num_models = P.shape[0]
    info = plsc.get_sparse_core_info()
    num_workers = info.num_cores * info.num_subcores
    bpw = batch // num_workers

    c_out = pl.pallas_call(
        _proj_kernel,
        out_shape=jax.ShapeDtypeStruct((1, 16), jnp.float32),
    )(prompt_embed, Wp, bp.reshape(1, DIM), Wc.reshape(1, DIM))

    pt = P.T
    n_blocks = pl.cdiv(num_models, COL_BLOCK)
    r = pl.pallas_call(
        _reduce_kernel,
        grid=(n_blocks,),
        in_specs=[
            pl.BlockSpec((DIM, COL_BLOCK), lambda i: (0, i)),
            pl.BlockSpec((DIM, 1), lambda i: (0, 0)),
            pl.BlockSpec((1, 16), lambda i: (0, 0)),
        ],
        out_specs=pl.BlockSpec((COL_BLOCK,), lambda i: (i,)),
        out_shape=jax.ShapeDtypeStruct((num_models,), jnp.float32),
    )(pt, Wc, c_out)

    ids = model_ids.astype(jnp.int32)
    out = _make_sc_kernel(batch, num_workers, bpw)(ids, r)
    return out


# speedup vs baseline: 5.9493x; 1.2108x over previous
import functools

import jax
import jax.numpy as jnp
from jax import lax
from jax.experimental import pallas as pl
from jax.experimental.pallas import tpu as pltpu
from jax.experimental.pallas import tpu_sc as plsc

DIM = 64
COL_BLOCK = 32768
IDX_CHUNK = 128


def _proj_kernel(prompt_ref, wp_ref, bp_ref, wc_ref, out_ref):
    pe = (
        jnp.dot(prompt_ref[...], wp_ref[...], preferred_element_type=jnp.float32)
        + bp_ref[...]
    )
    nsq = jnp.sum(pe * pe)
    denom = jnp.maximum(jnp.sqrt(nsq), 1e-12)
    c = jnp.sum(pe * wc_ref[...]) / denom
    out_ref[...] = jnp.broadcast_to(c, (1, 16))


def _reduce_kernel(pt_ref, wc_ref, ones_ref, c_ref, out_ref):
    x = pt_ref[...]
    wc_t = wc_ref[...]
    ones = ones_ref[...]
    dot = jax.lax.dot_general(
        wc_t, x, (((1,), (0,)), ((), ())), preferred_element_type=jnp.float32
    )
    nsq = jax.lax.dot_general(
        ones, x * x, (((1,), (0,)), ((), ())),
        preferred_element_type=jnp.float32,
    )
    denom = jnp.maximum(jnp.sqrt(nsq), 1e-12)
    out_ref[...] = (dot / denom - c_ref[0, 0]).reshape(-1)


def _make_sc_kernel(batch, num_workers, bpw):
    n_chunks = bpw // IDX_CHUNK
    mesh = plsc.VectorSubcoreMesh(core_axis_name="c", subcore_axis_name="s")

    @functools.partial(
        pl.kernel,
        mesh=mesh,
        out_type=jax.ShapeDtypeStruct((batch,), jnp.float32),
        compiler_params=pltpu.CompilerParams(
            needs_layout_passes=False, use_tc_tiling_on_sc=False
        ),
        scratch_types=[
            [pltpu.VMEM((IDX_CHUNK,), jnp.int32) for _ in range(n_chunks)],
            pltpu.VMEM((bpw,), jnp.float32),
            pltpu.SemaphoreType.DMA,
        ],
    )
    def sc_kernel(ids_hbm, r_hbm, out_hbm, idx_vs, out_v, sem):
        wid = lax.axis_index("s") * 2 + lax.axis_index("c")
        base = wid * bpw
        for k in range(n_chunks):
            pltpu.sync_copy(
                ids_hbm.at[pl.ds(base + k * IDX_CHUNK, IDX_CHUNK)], idx_vs[k]
            )
        copies = []
        for k in range(n_chunks):
            copies.append(
                pltpu.async_copy(
                    r_hbm.at[idx_vs[k]],
                    out_v.at[pl.ds(k * IDX_CHUNK, IDX_CHUNK)],
                    sem,
                )
            )
        for cp in copies:
            cp.wait()
        pltpu.sync_copy(out_v, out_hbm.at[pl.ds(base, bpw)])

    return sc_kernel


def kernel(model_ids, prompt_embed, P, Wp, bp, Wc):
    batch = model_ids.shape[0]
    num_models = P.shape[0]
    info = plsc.get_sparse_core_info()
    num_workers = info.num_cores * info.num_subcores
    bpw = batch // num_workers

    c_out = pl.pallas_call(
        _proj_kernel,
        out_shape=jax.ShapeDtypeStruct((1, 16), jnp.float32),
    )(prompt_embed, Wp, bp.reshape(1, DIM), Wc.reshape(1, DIM))

    pt = P.T
    n_blocks = pl.cdiv(num_models, COL_BLOCK)
    r = pl.pallas_call(
        _reduce_kernel,
        grid=(n_blocks,),
        in_specs=[
            pl.BlockSpec((DIM, COL_BLOCK), lambda i: (0, i)),
            pl.BlockSpec((1, DIM), lambda i: (0, 0)),
            pl.BlockSpec((1, DIM), lambda i: (0, 0)),
            pl.BlockSpec((1, 16), lambda i: (0, 0)),
        ],
        out_specs=pl.BlockSpec((COL_BLOCK,), lambda i: (i,)),
        out_shape=jax.ShapeDtypeStruct((num_models,), jnp.float32),
    )(pt, Wc.reshape(1, DIM), jnp.ones((1, DIM), jnp.float32), c_out)

    ids = model_ids.astype(jnp.int32)
    out = _make_sc_kernel(batch, num_workers, bpw)(ids, r)
    return out


# speedup vs baseline: 5.9822x; 1.0055x over previous
import functools

import jax
import jax.numpy as jnp
from jax import lax
from jax.experimental import pallas as pl
from jax.experimental.pallas import tpu as pltpu
from jax.experimental.pallas import tpu_sc as plsc

DIM = 64
COL_BLOCK = 65536
IDX_CHUNK = 128


def _proj_kernel(prompt_ref, wp_ref, bp_ref, wc_ref, out_ref):
    pe = (
        jnp.dot(prompt_ref[...], wp_ref[...], preferred_element_type=jnp.float32)
        + bp_ref[...]
    )
    nsq = jnp.sum(pe * pe)
    denom = jnp.maximum(jnp.sqrt(nsq), 1e-12)
    c = jnp.sum(pe * wc_ref[...]) / denom
    out_ref[...] = jnp.broadcast_to(c, (1, 16))


def _reduce_kernel(pt_ref, wc_ref, ones_ref, c_ref, out_ref):
    x = pt_ref[...]
    wc_t = wc_ref[...]
    ones = ones_ref[...]
    dot = jax.lax.dot_general(
        wc_t, x, (((1,), (0,)), ((), ())), preferred_element_type=jnp.float32
    )
    nsq = jax.lax.dot_general(
        ones, x * x, (((1,), (0,)), ((), ())),
        preferred_element_type=jnp.float32,
    )
    denom = jnp.maximum(jnp.sqrt(nsq), 1e-12)
    out_ref[...] = (dot / denom - c_ref[0, 0]).reshape(-1)


def _make_sc_kernel(batch, num_workers, bpw):
    n_chunks = bpw // IDX_CHUNK
    mesh = plsc.VectorSubcoreMesh(core_axis_name="c", subcore_axis_name="s")

    @functools.partial(
        pl.kernel,
        mesh=mesh,
        out_type=jax.ShapeDtypeStruct((batch,), jnp.float32),
        compiler_params=pltpu.CompilerParams(
            needs_layout_passes=False, use_tc_tiling_on_sc=False
        ),
        scratch_types=[
            [pltpu.VMEM((IDX_CHUNK,), jnp.int32) for _ in range(n_chunks)],
            pltpu.VMEM((bpw,), jnp.float32),
            pltpu.SemaphoreType.DMA,
        ],
    )
    def sc_kernel(ids_hbm, r_hbm, out_hbm, idx_vs, out_v, sem):
        wid = lax.axis_index("s") * 2 + lax.axis_index("c")
        base = wid * bpw
        for k in range(n_chunks):
            pltpu.sync_copy(
                ids_hbm.at[pl.ds(base + k * IDX_CHUNK, IDX_CHUNK)], idx_vs[k]
            )
        copies = []
        for k in range(n_chunks):
            copies.append(
                pltpu.async_copy(
                    r_hbm.at[idx_vs[k]],
                    out_v.at[pl.ds(k * IDX_CHUNK, IDX_CHUNK)],
                    sem,
                )
            )
        for cp in copies:
            cp.wait()
        pltpu.sync_copy(out_v, out_hbm.at[pl.ds(base, bpw)])

    return sc_kernel


def kernel(model_ids, prompt_embed, P, Wp, bp, Wc):
    batch = model_ids.shape[0]
    num_models = P.shape[0]
    info = plsc.get_sparse_core_info()
    num_workers = info.num_cores * info.num_subcores
    bpw = batch // num_workers

    c_out = pl.pallas_call(
        _proj_kernel,
        out_shape=jax.ShapeDtypeStruct((1, 16), jnp.float32),
    )(prompt_embed, Wp, bp.reshape(1, DIM), Wc.reshape(1, DIM))

    pt = P.T
    n_blocks = pl.cdiv(num_models, COL_BLOCK)
    r = pl.pallas_call(
        _reduce_kernel,
        grid=(n_blocks,),
        in_specs=[
            pl.BlockSpec((DIM, COL_BLOCK), lambda i: (0, i)),
            pl.BlockSpec((1, DIM), lambda i: (0, 0)),
            pl.BlockSpec((1, DIM), lambda i: (0, 0)),
            pl.BlockSpec((1, 16), lambda i: (0, 0)),
        ],
        out_specs=pl.BlockSpec((COL_BLOCK,), lambda i: (i,)),
        out_shape=jax.ShapeDtypeStruct((num_models,), jnp.float32),
    )(pt, Wc.reshape(1, DIM), jnp.ones((1, DIM), jnp.float32), c_out)

    ids = model_ids.astype(jnp.int32)
    out = _make_sc_kernel(batch, num_workers, bpw)(ids, r)
    return out
